# baseline (speedup 1.0000x reference)
import functools

import jax
import jax.numpy as jnp
from jax import lax
from jax.experimental import pallas as pl
from jax.experimental.pallas import tpu as pltpu
from jax.experimental.pallas import tpu_sc as plsc

CUTOFF = 12.0
CUTON = 0.8 * CUTOFF
KE = 7.199822675975274

_IW = 1.0 / (CUTOFF - CUTON)
_T0 = -CUTON / (CUTOFF - CUTON) - 0.5
_KA = 3.599911337987637
_Q0 = -11.133901964648322
_Q1 = 15.736317154692193
_NC2 = -1.0 / (CUTOFF * CUTOFF)
_C2A = 2.0 / CUTOFF

NC = 2
NS = 16
L = 16
NW = NC * NS


def _make_kernel(E, N, CH):
    per_w = E // (NW * CH)
    mesh = plsc.VectorSubcoreMesh(core_axis_name="c", subcore_axis_name="s")

    @functools.partial(
        pl.kernel,
        out_type=jax.ShapeDtypeStruct((E,), jnp.float32),
        mesh=mesh,
        compiler_params=pltpu.CompilerParams(needs_layout_passes=False),
        scratch_types=[
            pltpu.VMEM((N,), jnp.float32),
            pltpu.VMEM((CH,), jnp.int32),
            pltpu.VMEM((CH,), jnp.int32),
            pltpu.VMEM((CH,), jnp.int32),
            pltpu.VMEM((CH,), jnp.int32),
            pltpu.VMEM((CH,), jnp.float32),
            pltpu.VMEM((CH,), jnp.float32),
            pltpu.VMEM((CH,), jnp.float32),
            pltpu.VMEM((CH,), jnp.float32),
            pltpu.SemaphoreType.DMA,
            pltpu.SemaphoreType.DMA,
            pltpu.SemaphoreType.DMA,
            pltpu.SemaphoreType.DMA,
        ],
    )
    def k(dist_hbm, q_hbm, idxu_hbm, idxv_hbm, out_hbm,
          q_v, u0, u1, v0, v1, d0, d1, o0, o1, si0, si1, so0, so1):
        wid = lax.axis_index("s") * NC + lax.axis_index("c")
        base = wid * (per_w * CH)
        ubuf = (u0, u1)
        vbuf = (v0, v1)
        dbuf = (d0, d1)
        obuf = (o0, o1)
        sin_ = (si0, si1)
        sout = (so0, so1)

        pltpu.sync_copy(q_hbm, q_v)

        def issue_in(ci, b):
            off = base + ci * CH
            pltpu.async_copy(idxu_hbm.at[pl.ds(off, CH)], ubuf[b], sin_[b])
            pltpu.async_copy(idxv_hbm.at[pl.ds(off, CH)], vbuf[b], sin_[b])
            pltpu.async_copy(dist_hbm.at[pl.ds(off, CH)], dbuf[b], sin_[b])

        def wait_in(b):
            pltpu.make_async_copy(idxu_hbm.at[pl.ds(0, CH)], ubuf[b], sin_[b]).wait()
            pltpu.make_async_copy(idxv_hbm.at[pl.ds(0, CH)], vbuf[b], sin_[b]).wait()
            pltpu.make_async_copy(dist_hbm.at[pl.ds(0, CH)], dbuf[b], sin_[b]).wait()

        def compute(b):
            @plsc.parallel_loop(0, CH // L, unroll=8)
            def _(i):
                s = i * L
                d = dbuf[b][pl.ds(s, L)]
                qi = plsc.load_gather(q_v, [ubuf[b][pl.ds(s, L)]])
                qj = plsc.load_gather(q_v, [vbuf[b][pl.ds(s, L)]])
                t = jnp.maximum(d, CUTON) * _IW + _T0
                u = t * t
                p = _Q1 * u + _Q0
                w = t * p + _KA
                g = 1.0 / d - (d * _NC2 + _C2A)
                obuf[b][pl.ds(s, L)] = (qi * qj) * g * w

        def outer(cc, _):
            for b in (0, 1):
                ci = cc * 2 + b
                wait_in(b)
                if b == 0:
                    issue_in(ci + 1, 1)
                else:
                    @pl.when(cc < per_w // 2 - 1)
                    def _():
                        issue_in(ci + 1, 0)

                @pl.when(cc >= 1)
                def _():
                    pltpu.make_async_copy(
                        obuf[b], out_hbm.at[pl.ds(0, CH)], sout[b]).wait()

                compute(b)
                off = base + ci * CH
                pltpu.async_copy(obuf[b], out_hbm.at[pl.ds(off, CH)], sout[b])
            return 0

        issue_in(0, 0)
        lax.fori_loop(0, per_w // 2, outer, 0)
        for b in (0, 1):
            pltpu.make_async_copy(obuf[b], out_hbm.at[pl.ds(0, CH)], sout[b]).wait()

    return k


def kernel(mlmm_distances, mlmm_atomic_charges, mlmm_idxu, mlmm_idxv):
    E = mlmm_distances.shape[0]
    N = mlmm_atomic_charges.shape[0]
    k = _make_kernel(E, N, 2000)
    return k(mlmm_distances, mlmm_atomic_charges,
             mlmm_idxu.astype(jnp.int32), mlmm_idxv.astype(jnp.int32))


# speedup vs baseline: 1.2324x; 1.2324x over previous
import functools

import jax
import jax.numpy as jnp
from jax import lax
from jax.experimental import pallas as pl
from jax.experimental.pallas import tpu as pltpu
from jax.experimental.pallas import tpu_sc as plsc

CUTOFF = 12.0
CUTON = 0.8 * CUTOFF
KE = 7.199822675975274

_IW = 1.0 / (CUTOFF - CUTON)
_T0 = -CUTON / (CUTOFF - CUTON) - 0.5
_KA = 3.599911337987637
_Q0 = -11.133901964648322
_Q1 = 15.736317154692193
_NC2 = -1.0 / (CUTOFF * CUTOFF)
_C2A = 2.0 / CUTOFF

NC = 2
NS = 16
L = 16
NW = NC * NS


def _make_kernel(E, N, CH):
    per_w = E // NW
    n_full = per_w // CH
    tail = per_w - n_full * CH
    assert n_full % 2 == 0 and CH % L == 0 and tail % L == 0
    mesh = plsc.VectorSubcoreMesh(core_axis_name="c", subcore_axis_name="s")

    @functools.partial(
        pl.kernel,
        out_type=jax.ShapeDtypeStruct((E,), jnp.float32),
        mesh=mesh,
        compiler_params=pltpu.CompilerParams(needs_layout_passes=False),
        scratch_types=[
            pltpu.VMEM((N,), jnp.float32),
            pltpu.VMEM((CH,), jnp.int32),
            pltpu.VMEM((CH,), jnp.int32),
            pltpu.VMEM((CH,), jnp.int32),
            pltpu.VMEM((CH,), jnp.int32),
            pltpu.VMEM((CH,), jnp.float32),
            pltpu.VMEM((CH,), jnp.float32),
            pltpu.VMEM((CH,), jnp.float32),
            pltpu.VMEM((CH,), jnp.float32),
            pltpu.SemaphoreType.DMA,
            pltpu.SemaphoreType.DMA,
            pltpu.SemaphoreType.DMA,
            pltpu.SemaphoreType.DMA,
            pltpu.SemaphoreType.DMA,
        ],
    )
    def k(dist_hbm, q_hbm, idxu_hbm, idxv_hbm, out_hbm,
          q_v, u0, u1, v0, v1, d0, d1, o0, o1, si0, si1, so0, so1, sq):
        wid = lax.axis_index("s") * NC + lax.axis_index("c")
        base = wid * per_w
        ubuf = (u0, u1)
        vbuf = (v0, v1)
        dbuf = (d0, d1)
        obuf = (o0, o1)
        sin_ = (si0, si1)
        sout = (so0, so1)

        def issue_in(ci, b, n):
            off = base + ci * CH
            pltpu.async_copy(idxu_hbm.at[pl.ds(off, n)],
                             ubuf[b].at[pl.ds(0, n)], sin_[b])
            pltpu.async_copy(idxv_hbm.at[pl.ds(off, n)],
                             vbuf[b].at[pl.ds(0, n)], sin_[b])
            pltpu.async_copy(dist_hbm.at[pl.ds(off, n)],
                             dbuf[b].at[pl.ds(0, n)], sin_[b])

        def wait_in(b, n):
            pltpu.make_async_copy(dist_hbm.at[pl.ds(0, 3 * n)],
                                  q_v.at[pl.ds(0, 3 * n)], sin_[b]).wait()

        def wait_out(b, n):
            pltpu.make_async_copy(obuf[b].at[pl.ds(0, n)],
                                  out_hbm.at[pl.ds(0, n)], sout[b]).wait()

        def compute(b, n):
            @plsc.parallel_loop(0, n // L, unroll=8)
            def _(i):
                s = i * L
                d = dbuf[b][pl.ds(s, L)]
                qi = plsc.load_gather(q_v, [ubuf[b][pl.ds(s, L)]])
                qj = plsc.load_gather(q_v, [vbuf[b][pl.ds(s, L)]])
                t = jnp.maximum(d, CUTON) * _IW + _T0
                u = t * t
                p = _Q1 * u + _Q0
                w = t * p + _KA
                g = 1.0 / d - (d * _NC2 + _C2A)
                obuf[b][pl.ds(s, L)] = (qi * qj) * g * w

        cp_q = pltpu.async_copy(q_hbm, q_v, sq)
        issue_in(0, 0, CH)
        cp_q.wait()

        def outer(cc, _):
            for b in (0, 1):
                ci = cc * 2 + b
                wait_in(b, CH)
                if b == 0:
                    issue_in(ci + 1, 1, CH)
                else:
                    @pl.when(cc < n_full // 2 - 1)
                    def _():
                        issue_in(ci + 1, 0, CH)

                @pl.when(cc >= 1)
                def _():
                    wait_out(b, CH)

                compute(b, CH)
                off = base + ci * CH
                pltpu.async_copy(obuf[b], out_hbm.at[pl.ds(off, CH)], sout[b])
            return 0

        lax.fori_loop(0, n_full // 2, outer, 0)

        if tail:
            issue_in(n_full, 0, tail)
            wait_in(0, tail)
            wait_out(0, CH)
            compute(0, tail)
            off = base + n_full * CH
            pltpu.async_copy(obuf[0].at[pl.ds(0, tail)],
                             out_hbm.at[pl.ds(off, tail)], sout[0])
            wait_out(0, tail)
            wait_out(1, CH)
        else:
            wait_out(0, CH)
            wait_out(1, CH)

    return k


def kernel(mlmm_distances, mlmm_atomic_charges, mlmm_idxu, mlmm_idxv):
    E = mlmm_distances.shape[0]
    N = mlmm_atomic_charges.shape[0]
    k = _make_kernel(E, N, 3200)
    return k(mlmm_distances, mlmm_atomic_charges,
             mlmm_idxu.astype(jnp.int32), mlmm_idxv.astype(jnp.int32))


# speedup vs baseline: 1.3260x; 1.0759x over previous
import functools

import jax
import jax.numpy as jnp
from jax import lax
from jax.experimental import pallas as pl
from jax.experimental.pallas import tpu as pltpu
from jax.experimental.pallas import tpu_sc as plsc

CUTOFF = 12.0
CUTON = 0.8 * CUTOFF
KE = 7.199822675975274

_IW = 1.0 / (CUTOFF - CUTON)
_T0 = -CUTON / (CUTOFF - CUTON) - 0.5
_KA = 3.599911337987637
_Q0 = -11.133901964648322
_Q1 = 15.736317154692193
_NC2 = -1.0 / (CUTOFF * CUTOFF)
_C2A = 2.0 / CUTOFF

NC = 2
NS = 16
L = 16
NW = NC * NS


def _make_kernel(E, N, CH):
    per_w = E // NW
    n_full = per_w // CH
    tail = per_w - n_full * CH
    assert n_full % 2 == 0 and CH % L == 0 and tail % L == 0
    mesh = plsc.VectorSubcoreMesh(core_axis_name="c", subcore_axis_name="s")

    @functools.partial(
        pl.kernel,
        out_type=jax.ShapeDtypeStruct((E,), jnp.float32),
        mesh=mesh,
        compiler_params=pltpu.CompilerParams(needs_layout_passes=False),
        scratch_types=[
            pltpu.VMEM((N,), jnp.float32),
            pltpu.VMEM((CH,), jnp.int32),
            pltpu.VMEM((CH,), jnp.int32),
            pltpu.VMEM((CH,), jnp.int32),
            pltpu.VMEM((CH,), jnp.int32),
            pltpu.VMEM((CH,), jnp.float32),
            pltpu.VMEM((CH,), jnp.float32),
            pltpu.VMEM((CH,), jnp.float32),
            pltpu.VMEM((CH,), jnp.float32),
            pltpu.SemaphoreType.DMA,
            pltpu.SemaphoreType.DMA,
            pltpu.SemaphoreType.DMA,
            pltpu.SemaphoreType.DMA,
            pltpu.SemaphoreType.DMA,
        ],
    )
    def k(dist_hbm, q_hbm, idxu_hbm, idxv_hbm, out_hbm,
          q_v, u0, u1, v0, v1, d0, d1, o0, o1, si0, si1, so0, so1, sq):
        wid = lax.axis_index("s") * NC + lax.axis_index("c")
        base = wid * per_w
        ubuf = (u0, u1)
        vbuf = (v0, v1)
        dbuf = (d0, d1)
        obuf = (o0, o1)
        sin_ = (si0, si1)
        sout = (so0, so1)

        def issue_in(ci, b, n):
            off = base + ci * CH
            pltpu.async_copy(idxu_hbm.at[pl.ds(off, n)],
                             ubuf[b].at[pl.ds(0, n)], sin_[b])
            pltpu.async_copy(idxv_hbm.at[pl.ds(off, n)],
                             vbuf[b].at[pl.ds(0, n)], sin_[b])
            pltpu.async_copy(dist_hbm.at[pl.ds(off, n)],
                             dbuf[b].at[pl.ds(0, n)], sin_[b])

        def wait_in(b, n):
            pltpu.make_async_copy(dist_hbm.at[pl.ds(0, 3 * n)],
                                  q_v.at[pl.ds(0, 3 * n)], sin_[b]).wait()

        def wait_out(b, n):
            pltpu.make_async_copy(obuf[b].at[pl.ds(0, n)],
                                  out_hbm.at[pl.ds(0, n)], sout[b]).wait()

        def compute(b, n):
            @plsc.parallel_loop(0, n // L, unroll=8)
            def _(i):
                s = i * L
                d = dbuf[b][pl.ds(s, L)]
                qi = plsc.load_gather(q_v, [ubuf[b][pl.ds(s, L)]])
                qj = plsc.load_gather(q_v, [vbuf[b][pl.ds(s, L)]])
                t = jnp.maximum(d, CUTON) * _IW + _T0
                u = t * t
                p = _Q1 * u + _Q0
                w = t * p + _KA
                g = 1.0 / d - (d * _NC2 + _C2A)
                obuf[b][pl.ds(s, L)] = (qi * qj) * g * w

        cp_q = pltpu.async_copy(q_hbm, q_v, sq)
        issue_in(0, 0, CH)
        cp_q.wait()

        def outer(cc, _):
            for b in (0, 1):
                ci = cc * 2 + b
                wait_in(b, CH)
                if b == 0:
                    issue_in(ci + 1, 1, CH)
                else:
                    @pl.when(cc < n_full // 2 - 1)
                    def _():
                        issue_in(ci + 1, 0, CH)

                @pl.when(cc >= 1)
                def _():
                    wait_out(b, CH)

                compute(b, CH)
                off = base + ci * CH
                pltpu.async_copy(obuf[b], out_hbm.at[pl.ds(off, CH)], sout[b])
            return 0

        lax.fori_loop(0, n_full // 2, outer, 0)

        if tail:
            issue_in(n_full, 0, tail)
            wait_in(0, tail)
            wait_out(0, CH)
            compute(0, tail)
            off = base + n_full * CH
            pltpu.async_copy(obuf[0].at[pl.ds(0, tail)],
                             out_hbm.at[pl.ds(off, tail)], sout[0])
            wait_out(0, tail)
            wait_out(1, CH)
        else:
            wait_out(0, CH)
            wait_out(1, CH)

    return k


def kernel(mlmm_distances, mlmm_atomic_charges, mlmm_idxu, mlmm_idxv):
    E = mlmm_distances.shape[0]
    N = mlmm_atomic_charges.shape[0]
    k = _make_kernel(E, N, 3840)
    return k(mlmm_distances, mlmm_atomic_charges,
             mlmm_idxu.astype(jnp.int32), mlmm_idxv.astype(jnp.int32))


# speedup vs baseline: 1.4556x; 1.0977x over previous
import functools

import jax
import jax.numpy as jnp
from jax import lax
from jax.experimental import pallas as pl
from jax.experimental.pallas import tpu as pltpu
from jax.experimental.pallas import tpu_sc as plsc

CUTOFF = 12.0
CUTON = 0.8 * CUTOFF
KE = 7.199822675975274

_IW = 1.0 / (CUTOFF - CUTON)
_T0 = -CUTON / (CUTOFF - CUTON) - 0.5
_KA = 3.599911337987637
_Q0 = -11.133901964648322
_Q1 = 15.736317154692193
_NC2 = -1.0 / (CUTOFF * CUTOFF)
_C2A = 2.0 / CUTOFF

NC = 2
NS = 16
L = 16
NW = NC * NS


def _make_kernel(E, N, CH):
    per_w = E // NW
    n_full = per_w // CH
    tail = per_w - n_full * CH
    assert n_full % 2 == 0 and CH % L == 0 and tail % L == 0
    mesh = plsc.VectorSubcoreMesh(core_axis_name="c", subcore_axis_name="s")

    @functools.partial(
        pl.kernel,
        out_type=jax.ShapeDtypeStruct((E,), jnp.float32),
        mesh=mesh,
        compiler_params=pltpu.CompilerParams(needs_layout_passes=False),
        scratch_types=[
            pltpu.VMEM((N,), jnp.float32),
            pltpu.VMEM((CH,), jnp.int32),
            pltpu.VMEM((CH,), jnp.int32),
            pltpu.VMEM((CH,), jnp.int32),
            pltpu.VMEM((CH,), jnp.int32),
            pltpu.VMEM((CH,), jnp.float32),
            pltpu.VMEM((CH,), jnp.float32),
            pltpu.VMEM((CH,), jnp.float32),
            pltpu.VMEM((CH,), jnp.float32),
            pltpu.SemaphoreType.DMA,
            pltpu.SemaphoreType.DMA,
            pltpu.SemaphoreType.DMA,
            pltpu.SemaphoreType.DMA,
            pltpu.SemaphoreType.DMA,
        ],
    )
    def k(dist_hbm, q_hbm, idxu_hbm, idxv_hbm, out_hbm,
          q_v, u0, u1, v0, v1, d0, d1, o0, o1, si0, si1, so0, so1, sq):
        wid = lax.axis_index("s") * NC + lax.axis_index("c")
        base = wid * per_w
        ubuf = (u0, u1)
        vbuf = (v0, v1)
        dbuf = (d0, d1)
        obuf = (o0, o1)
        sin_ = (si0, si1)
        sout = (so0, so1)

        def issue_in(ci, b, n):
            off = base + ci * CH
            pltpu.async_copy(idxu_hbm.at[pl.ds(off, n)],
                             ubuf[b].at[pl.ds(0, n)], sin_[b])
            pltpu.async_copy(idxv_hbm.at[pl.ds(off, n)],
                             vbuf[b].at[pl.ds(0, n)], sin_[b])
            pltpu.async_copy(dist_hbm.at[pl.ds(off, n)],
                             dbuf[b].at[pl.ds(0, n)], sin_[b])

        def wait_in(b, n):
            pltpu.make_async_copy(dist_hbm.at[pl.ds(0, 3 * n)],
                                  q_v.at[pl.ds(0, 3 * n)], sin_[b]).wait()

        def wait_out(b, n):
            pltpu.make_async_copy(obuf[b].at[pl.ds(0, n)],
                                  out_hbm.at[pl.ds(0, n)], sout[b]).wait()

        def compute(b, n):
            @plsc.parallel_loop(0, n // L, unroll=8)
            def _(i):
                s = i * L
                d = dbuf[b][pl.ds(s, L)]
                qi = plsc.load_gather(q_v, [ubuf[b][pl.ds(s, L)]])
                qj = plsc.load_gather(q_v, [vbuf[b][pl.ds(s, L)]])
                t = jnp.maximum(d, CUTON) * _IW + _T0
                u = t * t
                p = _Q1 * u + _Q0
                w = t * p + _KA
                g = 1.0 / d - (d * _NC2 + _C2A)
                obuf[b][pl.ds(s, L)] = (qi * qj) * g * w

        cp_q = pltpu.async_copy(q_hbm, q_v, sq)
        issue_in(0, 0, CH)
        cp_q.wait()

        def outer(cc, _):
            for b in (0, 1):
                ci = cc * 2 + b
                if b == 0:
                    issue_in(ci + 1, 1, CH)
                else:
                    @pl.when(cc < n_full // 2 - 1)
                    def _():
                        issue_in(ci + 1, 0, CH)

                wait_in(b, CH)

                @pl.when(cc >= 1)
                def _():
                    wait_out(b, CH)

                compute(b, CH)
                off = base + ci * CH
                pltpu.async_copy(obuf[b], out_hbm.at[pl.ds(off, CH)], sout[b])
            return 0

        lax.fori_loop(0, n_full // 2, outer, 0)

        if tail:
            issue_in(n_full, 0, tail)
            wait_in(0, tail)
            wait_out(0, CH)
            compute(0, tail)
            off = base + n_full * CH
            pltpu.async_copy(obuf[0].at[pl.ds(0, tail)],
                             out_hbm.at[pl.ds(off, tail)], sout[0])
            wait_out(0, tail)
            wait_out(1, CH)
        else:
            wait_out(0, CH)
            wait_out(1, CH)

    return k


def kernel(mlmm_distances, mlmm_atomic_charges, mlmm_idxu, mlmm_idxv):
    E = mlmm_distances.shape[0]
    N = mlmm_atomic_charges.shape[0]
    k = _make_kernel(E, N, 3840)
    return k(mlmm_distances, mlmm_atomic_charges,
             mlmm_idxu.astype(jnp.int32), mlmm_idxv.astype(jnp.int32))


# speedup vs baseline: 1.4559x; 1.0002x over previous
import functools

import jax
import jax.numpy as jnp
from jax import lax
from jax.experimental import pallas as pl
from jax.experimental.pallas import tpu as pltpu
from jax.experimental.pallas import tpu_sc as plsc

CUTOFF = 12.0
CUTON = 0.8 * CUTOFF
KE = 7.199822675975274

_IW = 1.0 / (CUTOFF - CUTON)
_T0 = -CUTON / (CUTOFF - CUTON) - 0.5
_KA = 3.599911337987637
_Q0 = -11.133901964648322
_Q1 = 15.736317154692193
_NC2 = -1.0 / (CUTOFF * CUTOFF)
_C2A = 2.0 / CUTOFF

NC = 2
NS = 16
L = 16
NW = NC * NS


def _make_kernel(E, N, CH):
    per_w = E // NW
    n_full = per_w // CH
    tail = per_w - n_full * CH
    assert n_full % 2 == 0 and CH % L == 0 and tail % L == 0
    mesh = plsc.VectorSubcoreMesh(core_axis_name="c", subcore_axis_name="s")

    @functools.partial(
        pl.kernel,
        out_type=jax.ShapeDtypeStruct((E,), jnp.float32),
        mesh=mesh,
        compiler_params=pltpu.CompilerParams(needs_layout_passes=False),
        scratch_types=[
            pltpu.VMEM((N,), jnp.float32),
            pltpu.VMEM((CH,), jnp.int32),
            pltpu.VMEM((CH,), jnp.int32),
            pltpu.VMEM((CH,), jnp.int32),
            pltpu.VMEM((CH,), jnp.int32),
            pltpu.VMEM((CH,), jnp.float32),
            pltpu.VMEM((CH,), jnp.float32),
            pltpu.VMEM((CH,), jnp.float32),
            pltpu.VMEM((CH,), jnp.float32),
            pltpu.SemaphoreType.DMA,
            pltpu.SemaphoreType.DMA,
            pltpu.SemaphoreType.DMA,
            pltpu.SemaphoreType.DMA,
            pltpu.SemaphoreType.DMA,
        ],
    )
    def k(dist_hbm, q_hbm, idxu_hbm, idxv_hbm, out_hbm,
          q_v, u0, u1, v0, v1, d0, d1, o0, o1, si0, si1, so0, so1, sq):
        wid = lax.axis_index("s") * NC + lax.axis_index("c")
        base = wid * per_w
        ubuf = (u0, u1)
        vbuf = (v0, v1)
        dbuf = (d0, d1)
        obuf = (o0, o1)
        sin_ = (si0, si1)
        sout = (so0, so1)

        def issue_in(ci, b, n):
            off = base + ci * CH
            pltpu.async_copy(idxu_hbm.at[pl.ds(off, n)],
                             ubuf[b].at[pl.ds(0, n)], sin_[b])
            pltpu.async_copy(idxv_hbm.at[pl.ds(off, n)],
                             vbuf[b].at[pl.ds(0, n)], sin_[b])
            pltpu.async_copy(dist_hbm.at[pl.ds(off, n)],
                             dbuf[b].at[pl.ds(0, n)], sin_[b])

        def wait_in(b, n):
            pltpu.make_async_copy(dist_hbm.at[pl.ds(0, 3 * n)],
                                  q_v.at[pl.ds(0, 3 * n)], sin_[b]).wait()

        def wait_out(b, n):
            pltpu.make_async_copy(obuf[b].at[pl.ds(0, n)],
                                  out_hbm.at[pl.ds(0, n)], sout[b]).wait()

        def compute(b, n):
            @plsc.parallel_loop(0, n // L, unroll=12)
            def _(i):
                s = i * L
                d = dbuf[b][pl.ds(s, L)]
                qi = plsc.load_gather(q_v, [ubuf[b][pl.ds(s, L)]])
                qj = plsc.load_gather(q_v, [vbuf[b][pl.ds(s, L)]])
                t = jnp.maximum(d, CUTON) * _IW + _T0
                u = t * t
                p = _Q1 * u + _Q0
                w = t * p + _KA
                g = 1.0 / d - (d * _NC2 + _C2A)
                obuf[b][pl.ds(s, L)] = (qi * qj) * g * w

        cp_q = pltpu.async_copy(q_hbm, q_v, sq)
        issue_in(0, 0, CH)
        cp_q.wait()

        def outer(cc, _):
            for b in (0, 1):
                ci = cc * 2 + b
                if b == 0:
                    issue_in(ci + 1, 1, CH)
                else:
                    @pl.when(cc < n_full // 2 - 1)
                    def _():
                        issue_in(ci + 1, 0, CH)

                wait_in(b, CH)

                @pl.when(cc >= 1)
                def _():
                    wait_out(b, CH)

                compute(b, CH)
                off = base + ci * CH
                pltpu.async_copy(obuf[b], out_hbm.at[pl.ds(off, CH)], sout[b])
            return 0

        lax.fori_loop(0, n_full // 2, outer, 0)

        if tail:
            issue_in(n_full, 0, tail)
            wait_in(0, tail)
            wait_out(0, CH)
            compute(0, tail)
            off = base + n_full * CH
            pltpu.async_copy(obuf[0].at[pl.ds(0, tail)],
                             out_hbm.at[pl.ds(off, tail)], sout[0])
            wait_out(0, tail)
            wait_out(1, CH)
        else:
            wait_out(0, CH)
            wait_out(1, CH)

    return k


def kernel(mlmm_distances, mlmm_atomic_charges, mlmm_idxu, mlmm_idxv):
    E = mlmm_distances.shape[0]
    N = mlmm_atomic_charges.shape[0]
    k = _make_kernel(E, N, 3840)
    return k(mlmm_distances, mlmm_atomic_charges,
             mlmm_idxu.astype(jnp.int32), mlmm_idxv.astype(jnp.int32))


# speedup vs baseline: 1.5203x; 1.0443x over previous
import functools

import jax
import jax.numpy as jnp
from jax import lax
from jax.experimental import pallas as pl
from jax.experimental.pallas import tpu as pltpu
from jax.experimental.pallas import tpu_sc as plsc

CUTOFF = 12.0
CUTON = 0.8 * CUTOFF
KE = 7.199822675975274

_IW = 1.0 / (CUTOFF - CUTON)
_T0 = -CUTON / (CUTOFF - CUTON) - 0.5
_KA = 3.599911337987637
_Q0 = -11.133901964648322
_Q1 = 15.736317154692193
_NC2 = -1.0 / (CUTOFF * CUTOFF)
_C2A = 2.0 / CUTOFF

NC = 2
NS = 16
L = 16
NW = NC * NS

_NIN = 3
_NOUT = 2


def _make_kernel(E, N, CH):
    per_w = E // NW
    n_full = per_w // CH
    tail = per_w - n_full * CH
    grp = _NIN * _NOUT
    assert n_full % grp == 0 and CH % L == 0 and tail % L == 0 and tail <= CH
    mesh = plsc.VectorSubcoreMesh(core_axis_name="c", subcore_axis_name="s")

    @functools.partial(
        pl.kernel,
        out_type=jax.ShapeDtypeStruct((E,), jnp.float32),
        mesh=mesh,
        compiler_params=pltpu.CompilerParams(needs_layout_passes=False),
        scratch_types=(
            [pltpu.VMEM((N,), jnp.float32)]
            + [pltpu.VMEM((CH,), jnp.int32) for _ in range(2 * _NIN)]
            + [pltpu.VMEM((CH,), jnp.float32) for _ in range(_NIN)]
            + [pltpu.VMEM((CH,), jnp.float32) for _ in range(_NOUT)]
            + [pltpu.SemaphoreType.DMA for _ in range(_NIN + _NOUT + 1)]
        ),
    )
    def k(dist_hbm, q_hbm, idxu_hbm, idxv_hbm, out_hbm, q_v, *bufs):
        ubuf = bufs[0:_NIN]
        vbuf = bufs[_NIN:2 * _NIN]
        dbuf = bufs[2 * _NIN:3 * _NIN]
        obuf = bufs[3 * _NIN:3 * _NIN + _NOUT]
        sems = bufs[3 * _NIN + _NOUT:]
        sin_ = sems[0:_NIN]
        sout = sems[_NIN:_NIN + _NOUT]
        sq = sems[_NIN + _NOUT]

        wid = lax.axis_index("s") * NC + lax.axis_index("c")
        base = wid * per_w

        def issue_in(ci, b, n):
            off = base + ci * CH
            pltpu.async_copy(idxu_hbm.at[pl.ds(off, n)],
                             ubuf[b].at[pl.ds(0, n)], sin_[b])
            pltpu.async_copy(idxv_hbm.at[pl.ds(off, n)],
                             vbuf[b].at[pl.ds(0, n)], sin_[b])
            pltpu.async_copy(dist_hbm.at[pl.ds(off, n)],
                             dbuf[b].at[pl.ds(0, n)], sin_[b])

        def wait_in(b, n):
            pltpu.make_async_copy(dist_hbm.at[pl.ds(0, 3 * n)],
                                  q_v.at[pl.ds(0, 3 * n)], sin_[b]).wait()

        def wait_out(b, n):
            pltpu.make_async_copy(obuf[b].at[pl.ds(0, n)],
                                  out_hbm.at[pl.ds(0, n)], sout[b]).wait()

        def compute(bi, bo, n):
            @plsc.parallel_loop(0, n // L, unroll=12)
            def _(i):
                s = i * L
                d = dbuf[bi][pl.ds(s, L)]
                qi = plsc.load_gather(q_v, [ubuf[bi][pl.ds(s, L)]])
                qj = plsc.load_gather(q_v, [vbuf[bi][pl.ds(s, L)]])
                t = jnp.maximum(d, CUTON) * _IW + _T0
                u = t * t
                p = _Q1 * u + _Q0
                w = t * p + _KA
                g = 1.0 / d - (d * _NC2 + _C2A)
                obuf[bo][pl.ds(s, L)] = (qi * qj) * g * w

        cp_q = pltpu.async_copy(q_hbm, q_v, sq)
        issue_in(0, 0, CH)
        issue_in(1, 1, CH)
        cp_q.wait()

        def outer(cc, _):
            for j in range(grp):
                ci = cc * grp + j
                bi = j % _NIN
                bo = j % _NOUT
                nxt = (j + 2) % _NIN
                if j < grp - 2:
                    issue_in(ci + 2, nxt, CH)
                else:
                    @pl.when(cc < n_full // grp - 1)
                    def _():
                        issue_in(ci + 2, nxt, CH)

                wait_in(bi, CH)

                @pl.when(ci >= _NOUT)
                def _():
                    wait_out(bo, CH)

                compute(bi, bo, CH)
                off = base + ci * CH
                pltpu.async_copy(obuf[bo], out_hbm.at[pl.ds(off, CH)], sout[bo])
            return 0

        lax.fori_loop(0, n_full // grp, outer, 0)

        if tail:
            bi = n_full % _NIN
            bo = n_full % _NOUT
            issue_in(n_full, bi, tail)
            wait_in(bi, tail)
            wait_out(bo, CH)
            compute(bi, bo, tail)
            off = base + n_full * CH
            pltpu.async_copy(obuf[bo].at[pl.ds(0, tail)],
                             out_hbm.at[pl.ds(off, tail)], sout[bo])
            wait_out(bo, tail)
            wait_out(1 - bo, CH)
        else:
            wait_out(0, CH)
            wait_out(1, CH)

    return k


def kernel(mlmm_distances, mlmm_atomic_charges, mlmm_idxu, mlmm_idxv):
    E = mlmm_distances.shape[0]
    N = mlmm_atomic_charges.shape[0]
    k = _make_kernel(E, N, 2560)
    return k(mlmm_distances, mlmm_atomic_charges,
             mlmm_idxu.astype(jnp.int32), mlmm_idxv.astype(jnp.int32))


# speedup vs baseline: 1.5242x; 1.0026x over previous
import functools

import jax
import jax.numpy as jnp
from jax import lax
from jax.experimental import pallas as pl
from jax.experimental.pallas import tpu as pltpu
from jax.experimental.pallas import tpu_sc as plsc

CUTOFF = 12.0
CUTON = 0.8 * CUTOFF
KE = 7.199822675975274

_IW = 1.0 / (CUTOFF - CUTON)
_T0 = -CUTON / (CUTOFF - CUTON) - 0.5
_KA = 3.599911337987637
_Q0 = -11.133901964648322
_Q1 = 15.736317154692193
_NC2 = -1.0 / (CUTOFF * CUTOFF)
_C2A = 2.0 / CUTOFF

NC = 2
NS = 16
L = 16
NW = NC * NS

_NIN = 3
_NOUT = 2


def _make_kernel(E, N, CH):
    per_w = E // NW
    n_full = per_w // CH
    tail = per_w - n_full * CH
    grp = _NIN * _NOUT
    assert n_full % grp == 0 and CH % L == 0 and tail % L == 0 and tail <= CH
    mesh = plsc.VectorSubcoreMesh(core_axis_name="c", subcore_axis_name="s")

    @functools.partial(
        pl.kernel,
        out_type=jax.ShapeDtypeStruct((E,), jnp.float32),
        mesh=mesh,
        compiler_params=pltpu.CompilerParams(needs_layout_passes=False),
        scratch_types=(
            [pltpu.VMEM((N,), jnp.float32)]
            + [pltpu.VMEM((CH,), jnp.int32) for _ in range(2 * _NIN)]
            + [pltpu.VMEM((CH,), jnp.float32) for _ in range(_NIN)]
            + [pltpu.VMEM((CH,), jnp.float32) for _ in range(_NOUT)]
            + [pltpu.SemaphoreType.DMA for _ in range(_NIN + _NOUT + 1)]
        ),
    )
    def k(dist_hbm, q_hbm, idxu_hbm, idxv_hbm, out_hbm, q_v, *bufs):
        ubuf = bufs[0:_NIN]
        vbuf = bufs[_NIN:2 * _NIN]
        dbuf = bufs[2 * _NIN:3 * _NIN]
        obuf = bufs[3 * _NIN:3 * _NIN + _NOUT]
        sems = bufs[3 * _NIN + _NOUT:]
        sin_ = sems[0:_NIN]
        sout = sems[_NIN:_NIN + _NOUT]
        sq = sems[_NIN + _NOUT]

        wid = lax.axis_index("s") * NC + lax.axis_index("c")
        base = wid * per_w

        def issue_in(ci, b, n):
            off = base + ci * CH
            pltpu.async_copy(idxu_hbm.at[pl.ds(off, n)],
                             ubuf[b].at[pl.ds(0, n)], sin_[b])
            pltpu.async_copy(idxv_hbm.at[pl.ds(off, n)],
                             vbuf[b].at[pl.ds(0, n)], sin_[b])
            pltpu.async_copy(dist_hbm.at[pl.ds(off, n)],
                             dbuf[b].at[pl.ds(0, n)], sin_[b])

        def wait_in(b, n):
            pltpu.make_async_copy(dist_hbm.at[pl.ds(0, 3 * n)],
                                  q_v.at[pl.ds(0, 3 * n)], sin_[b]).wait()

        def wait_out(b, n):
            pltpu.make_async_copy(obuf[b].at[pl.ds(0, n)],
                                  out_hbm.at[pl.ds(0, n)], sout[b]).wait()

        def compute(bi, bo, n):
            @plsc.parallel_loop(0, n // L, unroll=12)
            def _(i):
                s = i * L
                d = dbuf[bi][pl.ds(s, L)]
                qi = plsc.load_gather(q_v, [ubuf[bi][pl.ds(s, L)]])
                qj = plsc.load_gather(q_v, [vbuf[bi][pl.ds(s, L)]])
                t = jnp.maximum(d, CUTON) * _IW + _T0
                u = t * t
                p = _Q1 * u + _Q0
                w = t * p + _KA
                g = 1.0 / d - (d * _NC2 + _C2A)
                obuf[bo][pl.ds(s, L)] = (qi * qj) * g * w

        cp_q = pltpu.async_copy(q_hbm, q_v, sq)
        issue_in(0, 0, CH)
        issue_in(1, 1, CH)
        cp_q.wait()

        def outer(cc, _):
            for j in range(grp):
                ci = cc * grp + j
                bi = j % _NIN
                bo = j % _NOUT
                nxt = (j + 2) % _NIN
                if j < grp - 2:
                    issue_in(ci + 2, nxt, CH)
                else:
                    @pl.when(cc < n_full // grp - 1)
                    def _():
                        issue_in(ci + 2, nxt, CH)

                wait_in(bi, CH)

                @pl.when(ci >= _NOUT)
                def _():
                    wait_out(bo, CH)

                compute(bi, bo, CH)
                off = base + ci * CH
                pltpu.async_copy(obuf[bo], out_hbm.at[pl.ds(off, CH)], sout[bo])
            return 0

        lax.fori_loop(0, n_full // grp, outer, 0)

        if tail:
            bi = n_full % _NIN
            bo = n_full % _NOUT
            issue_in(n_full, bi, tail)
            wait_in(bi, tail)
            wait_out(bo, CH)
            compute(bi, bo, tail)
            off = base + n_full * CH
            pltpu.async_copy(obuf[bo].at[pl.ds(0, tail)],
                             out_hbm.at[pl.ds(off, tail)], sout[bo])
            wait_out(bo, tail)
            wait_out(1 - bo, CH)
        else:
            wait_out(0, CH)
            wait_out(1, CH)

    return k


def kernel(mlmm_distances, mlmm_atomic_charges, mlmm_idxu, mlmm_idxv):
    E = mlmm_distances.shape[0]
    N = mlmm_atomic_charges.shape[0]
    k = _make_kernel(E, N, 2560)
    idxu = mlmm_idxu if mlmm_idxu.dtype == jnp.int32 else mlmm_idxu.astype(jnp.int32)
    idxv = mlmm_idxv if mlmm_idxv.dtype == jnp.int32 else mlmm_idxv.astype(jnp.int32)
    return k(mlmm_distances, mlmm_atomic_charges, idxu, idxv)
